# baseline (speedup 1.0000x reference)
import functools

import jax
import jax.numpy as jnp
from jax import lax
from jax.experimental import pallas as pl
from jax.experimental.pallas import tpu as pltpu
from jax.experimental.pallas import tpu_sc as plsc

_B = 16384
_NU = 100000
_D = 64
_LAMDA = 0.5
_LAMDA_T = 0.25
_NC = 2
_NS = 16
_NW = _NC * _NS
_BT = _B // _NS
_WG = _BT // 16
_NUP = 102400
_ZSH = _NUP // _NS
_BPW2 = 25
_NBLK = _NU // 128
_TAIL0 = _NBLK * 128
_BCH = 125


def _rsqrt(x):
    i = lax.bitcast_convert_type(x, jnp.int32)
    i = jnp.int32(0x5F3759DF) - (i >> 1)
    y = lax.bitcast_convert_type(i, jnp.float32)
    for _ in range(3):
        y = y * (1.5 - 0.5 * x * y * y)
    return y


def _body(lp_hbm, iu_hbm, tu_hbm, du_hbm, di_hbm, bu_hbm, bi_hbm,
          pqut_hbm, pqit_hbm, ywut_hbm, ywit_hbm,
          pqu_hbm, pqi_hbm, ywu_hbm, ywi_hbm, rdeg_hbm, tdeg_hbm,
          out_hbm,
          idxu_v, idxi_v, iu_v, tu_v, lp_v, rdeg_v, tdeg_v,
          wpu_s, wwv_s, wbu_s, wit_s, zed_v,
          bufa, bufb, bbuf_v, bwsl_v, tpu_w, twv_w, tbu_w, tit_w,
          tbuf, wtail_v, tacc_v, stage_v,
          Wpu_sh, Wwv_sh, Wbu_sh, Wit_sh,
          sem_a, sem_ba, sem_bb, sem_t):
    cid = lax.axis_index("c")
    sid = lax.axis_index("s")
    wid = sid * _NC + cid
    lane = lax.iota(jnp.int32, 16)
    zero = jnp.zeros((16,), jnp.float32)

    tb = sid * _BT
    pltpu.sync_copy(du_hbm.at[pl.ds(tb, _BT)], idxu_v)
    pltpu.sync_copy(di_hbm.at[pl.ds(tb, _BT)], idxi_v)
    pltpu.sync_copy(iu_hbm.at[pl.ds(tb, _BT)], iu_v)
    pltpu.sync_copy(tu_hbm.at[pl.ds(tb, _BT)], tu_v)
    pltpu.sync_copy(lp_hbm.at[pl.ds(tb, _BT)], lp_v)
    g_rd = pltpu.async_copy(rdeg_hbm.at[idxi_v], rdeg_v, sem_a)
    g_td = pltpu.async_copy(tdeg_hbm.at[idxu_v], tdeg_v, sem_a)

    def zbody(z, c):
        zed_v[pl.ds(z * 16, 16)] = zero
        return c
    lax.fori_loop(0, _ZSH // 16, zbody, 0)
    for W_sh in (Wpu_sh, Wwv_sh, Wbu_sh, Wit_sh):
        pltpu.sync_copy(zed_v, W_sh.at[pl.ds(sid * _ZSH, _ZSH)])

    g_rd.wait()
    g_td.wait()

    def wgroup(g, carry):
        lnk = carry
        sl = pl.ds(g * 16, 16)
        iu = iu_v[sl]
        tu = tu_v[sl]
        lpv = lp_v[sl]
        rdeg = rdeg_v[sl]
        tdeg = tdeg_v[sl]
        uj = jnp.where(rdeg > 0,
                       _rsqrt(jnp.maximum(rdeg.astype(jnp.float32), 1.0)), 0.0)
        tv = jnp.where(tdeg > 0,
                       _rsqrt(jnp.maximum(tdeg.astype(jnp.float32), 1.0)), 0.0)
        wbu_s[sl] = _LAMDA * iu
        wpu_s[sl] = _LAMDA * iu + _LAMDA_T * tu
        wwv_s[sl] = _LAMDA_T * tv
        wit_s[sl] = _LAMDA * uj
        d = lpv - 1.0
        return lnk + d * d

    lnk = lax.fori_loop(0, _WG, wgroup, zero)

    plsc.subcore_barrier()
    pltpu.sync_copy(wpu_s, Wpu_sh.at[idxu_v], add=True)
    pltpu.sync_copy(wwv_s, Wwv_sh.at[idxu_v], add=True)
    pltpu.sync_copy(wbu_s, Wbu_sh.at[idxu_v], add=True)
    pltpu.sync_copy(wit_s, Wit_sh.at[idxi_v], add=True)
    plsc.subcore_barrier()

    terr = wid * (_BPW2 * 128)
    pltpu.sync_copy(Wpu_sh.at[pl.ds(terr, _BPW2 * 128)], tpu_w)
    pltpu.sync_copy(Wwv_sh.at[pl.ds(terr, _BPW2 * 128)], twv_w)
    pltpu.sync_copy(Wbu_sh.at[pl.ds(terr, _BPW2 * 128)], tbu_w)
    pltpu.sync_copy(Wit_sh.at[pl.ds(terr, _BPW2 * 128)], tit_w)

    nblk = jnp.minimum(_BPW2, _NBLK - wid * _BPW2)

    def issue_half(tblt, h, blk, buf, sem):
        pltpu.async_copy(
            tblt.at[pl.ds(h * 32, 32), pl.ds(blk * 128, 128)], buf, sem)

    def wait_half(tblt, buf, sem):
        pltpu.make_async_copy(
            tblt.at[pl.ds(0, 32), pl.ds(0, 128)], buf, sem).wait()

    def scan_table(tblt, w_ref, reg):
        blk0 = wid * _BPW2
        issue_half(tblt, 0, blk0, bufa, sem_ba)
        issue_half(tblt, 1, blk0, bufb, sem_bb)

        def cpass(buf, wvs, acc):
            def cbody(c2, a):
                row = jnp.full((16,), c2, jnp.int32)
                out = []
                for q in range(8):
                    v = plsc.load_gather(buf, [row, lane + q * 16])
                    out.append(a[q] + wvs[q] * (v * v))
                return tuple(out)
            return lax.fori_loop(0, 32, cbody, acc)

        def mbody(m, acc):
            blk = blk0 + m
            wvs = [w_ref[pl.ds(m * 128 + q * 16, 16)] for q in range(8)]
            wait_half(tblt, bufa, sem_ba)
            acc = cpass(bufa, wvs, acc)

            @pl.when(m + 1 < nblk)
            def _():
                issue_half(tblt, 0, blk + 1, bufa, sem_ba)

            wait_half(tblt, bufb, sem_bb)
            acc = cpass(bufb, wvs, acc)

            @pl.when(m + 1 < nblk)
            def _():
                issue_half(tblt, 1, blk + 1, bufb, sem_bb)

            return acc

        acc = lax.fori_loop(0, nblk, mbody, (zero,) * 8)
        for q in range(8):
            reg = reg + acc[q]
        return reg

    reg = zero
    reg = scan_table(pqut_hbm, tpu_w, reg)
    reg = scan_table(ywut_hbm, twv_w, reg)
    reg = scan_table(pqit_hbm, tit_w, reg)
    reg = scan_table(ywit_hbm, tit_w, reg)

    def bchunk(j, acc):
        ch = wid + 32 * j
        chc = jnp.minimum(ch, _BCH - 1)
        valid = jnp.where(ch < _BCH, 1.0, 0.0)
        off = chc * 800
        pltpu.sync_copy(bu_hbm.at[pl.ds(off, 800)], bbuf_v)
        pltpu.sync_copy(Wbu_sh.at[pl.ds(off, 800)], bwsl_v)

        def bsum(i, a):
            v = bbuf_v[pl.ds(i * 16, 16)]
            w = bwsl_v[pl.ds(i * 16, 16)]
            return a + w * (v * v)
        a1 = lax.fori_loop(0, 50, bsum, zero)

        pltpu.sync_copy(bi_hbm.at[pl.ds(off, 800)], bbuf_v)
        pltpu.sync_copy(Wit_sh.at[pl.ds(off, 800)], bwsl_v)
        a2 = lax.fori_loop(0, 50, bsum, zero)
        return acc + valid * (a1 + a2)

    reg = reg + lax.fori_loop(0, 4, bchunk, zero)

    tacc_v[...] = zero

    @pl.when(wid == 0)
    def _tail():
        tail = zero
        for tbl, W_sh in ((pqu_hbm, Wpu_sh), (ywu_hbm, Wwv_sh),
                          (pqi_hbm, Wit_sh), (ywi_hbm, Wit_sh)):
            def tissue(r, c):
                pltpu.async_copy(tbl.at[pl.ds(_TAIL0 + r, 1)],
                                 tbuf.at[pl.ds(r, 1)], sem_t)
                return c
            lax.fori_loop(0, 32, tissue, 0)
            pltpu.make_async_copy(tbl.at[pl.ds(0, 32)], tbuf, sem_t).wait()
            pltpu.sync_copy(W_sh.at[pl.ds(_TAIL0, 32)], wtail_v)
            w0 = wtail_v[pl.ds(0, 16)]
            w1 = wtail_v[pl.ds(16, 16)]
            for j in range(32):
                w = w0[j] if j < 16 else w1[j - 16]
                rows = jnp.full((16,), j, jnp.int32)
                for q in range(4):
                    v = plsc.load_gather(tbuf, [rows, lane + q * 16])
                    tail = tail + w * (v * v)
        tacc_v[...] = tail

    reg = reg + tacc_v[...]

    stage_v[...] = reg
    pltpu.sync_copy(stage_v, out_hbm.at[pl.ds(wid * 32, 16)])
    stage_v[...] = lnk
    pltpu.sync_copy(stage_v, out_hbm.at[pl.ds(wid * 32 + 16, 16)])


_run = functools.partial(
    pl.kernel,
    mesh=plsc.VectorSubcoreMesh(core_axis_name="c", subcore_axis_name="s"),
    out_type=jax.ShapeDtypeStruct((_NW * 32,), jnp.float32),
    compiler_params=pltpu.CompilerParams(
        use_tc_tiling_on_sc=True, needs_layout_passes=False),
    scratch_types=[
        pltpu.VMEM((_BT,), jnp.int32),
        pltpu.VMEM((_BT,), jnp.int32),
        pltpu.VMEM((_BT,), jnp.float32),
        pltpu.VMEM((_BT,), jnp.float32),
        pltpu.VMEM((_BT,), jnp.float32),
        pltpu.VMEM((_BT,), jnp.int32),
        pltpu.VMEM((_BT,), jnp.int32),
        pltpu.VMEM((_BT,), jnp.float32),
        pltpu.VMEM((_BT,), jnp.float32),
        pltpu.VMEM((_BT,), jnp.float32),
        pltpu.VMEM((_BT,), jnp.float32),
        pltpu.VMEM((_ZSH,), jnp.float32),
        pltpu.VMEM((32, 128), jnp.float32),
        pltpu.VMEM((32, 128), jnp.float32),
        pltpu.VMEM((800,), jnp.float32),
        pltpu.VMEM((800,), jnp.float32),
        pltpu.VMEM((_BPW2 * 128,), jnp.float32),
        pltpu.VMEM((_BPW2 * 128,), jnp.float32),
        pltpu.VMEM((_BPW2 * 128,), jnp.float32),
        pltpu.VMEM((_BPW2 * 128,), jnp.float32),
        pltpu.VMEM((32, _D), jnp.float32),
        pltpu.VMEM((32,), jnp.float32),
        pltpu.VMEM((16,), jnp.float32),
        pltpu.VMEM((16,), jnp.float32),
        pltpu.VMEM_SHARED((_NUP,), jnp.float32),
        pltpu.VMEM_SHARED((_NUP,), jnp.float32),
        pltpu.VMEM_SHARED((_NUP,), jnp.float32),
        pltpu.VMEM_SHARED((_NUP,), jnp.float32),
        pltpu.SemaphoreType.DMA,
        pltpu.SemaphoreType.DMA,
        pltpu.SemaphoreType.DMA,
        pltpu.SemaphoreType.DMA,
    ],
)(_body)


def kernel(link_pred, bias_user, bias_item, p_q_user, p_q_item, y_w_user,
           y_w_item, I_u_factor, T_u_factor, dst_user, dst_item,
           rated_by_deg, trusted_by_deg):
    out = _run(
        link_pred,
        I_u_factor.reshape(-1),
        T_u_factor.reshape(-1),
        dst_user.astype(jnp.int32),
        dst_item.astype(jnp.int32),
        bias_user.reshape(-1),
        bias_item.reshape(-1),
        p_q_user.T, p_q_item.T, y_w_user.T, y_w_item.T,
        p_q_user, p_q_item, y_w_user, y_w_item,
        rated_by_deg.astype(jnp.int32),
        trusted_by_deg.astype(jnp.int32),
    )
    o = out.reshape(_NW, 2, 16)
    reg_loss = jnp.sum(o[:, 0, :]) / _B
    link_loss = _LAMDA_T * (jnp.sum(o[:, 1, :]) / (_NC * _B))
    return (reg_loss, link_loss)
